# Initial kernel scaffold; baseline (speedup 1.0000x reference)
import jax
import jax.numpy as jnp
from jax.experimental import pallas as pl


def kernel(x, tok_table, pos_table):
    raise NotImplementedError("write your pallas kernel here")


# speedup vs baseline: 1.1441x; 1.1441x over previous
import functools

import jax
import jax.numpy as jnp
from jax import lax
from jax.experimental import pallas as pl
from jax.experimental.pallas import tpu as pltpu
from jax.experimental.pallas import tpu_sc as plsc

_NUM_CORES = 2
_NUM_SUBCORES = 16
_NW = _NUM_CORES * _NUM_SUBCORES
_LANES = 16


@functools.lru_cache(maxsize=None)
def _make_kernel(B, T, D, V):
    assert T % _NW == 0 and D % _LANES == 0
    tpw = T // _NW
    groups = D // _LANES

    mesh = plsc.VectorSubcoreMesh(core_axis_name="c", subcore_axis_name="s")

    @functools.partial(
        pl.kernel,
        mesh=mesh,
        out_type=jax.ShapeDtypeStruct((B * T, D), jnp.float32),
        scratch_types=[
            pltpu.VMEM((tpw,), jnp.int32),
            pltpu.VMEM((tpw, D), jnp.float32),
            pltpu.VMEM((tpw, D), jnp.float32),
            pltpu.SemaphoreType.DMA,
        ],
    )
    def emb(x_hbm, tok_hbm, pos_hbm, out_hbm, idx_v, rows_v, pos_v, sem):
        wid = lax.axis_index("s") * _NUM_CORES + lax.axis_index("c")
        t0 = wid * tpw
        pltpu.sync_copy(pos_hbm.at[pl.ds(t0, tpw)], pos_v)
        for b in range(B):
            base = b * T + t0
            pltpu.sync_copy(x_hbm.at[pl.ds(base, tpw)], idx_v)
            pltpu.async_copy(tok_hbm.at[idx_v], rows_v, sem).wait()

            def row_add(r, carry):
                for g in range(groups):
                    sl = pl.ds(g * _LANES, _LANES)
                    plsc.addupdate(rows_v.at[r, sl], pos_v[r, sl])
                return carry

            lax.fori_loop(0, tpw, row_add, 0)
            pltpu.sync_copy(rows_v, out_hbm.at[pl.ds(base, tpw)])

    return emb


def kernel(x, tok_table, pos_table):
    B, T = x.shape
    V, D = tok_table.shape
    emb = _make_kernel(B, T, D, V)
    out = emb(x.reshape(-1).astype(jnp.int32), tok_table, pos_table)
    return out.reshape(B, T, D)
